# baseline (speedup 1.0000x reference)
import jax
import jax.numpy as jnp
from jax import lax
from jax.experimental import pallas as pl
from jax.experimental.pallas import tpu as pltpu
from jax.experimental.pallas import tpu_sc as plsc

B = 16384
D = 256
NW = 32
ROWS_PER_W = B // NW
G = 128
NGROUPS = ROWS_PER_W // G
SUB = G // 16


def _worker_body(x_hbm, map_hbm, out_hbm, buf0, buf1, map_v, out_v,
                 sem0, sem1, osem):
    wid = lax.axis_index("s") * 2 + lax.axis_index("c")
    row0 = wid * ROWS_PER_W

    pltpu.sync_copy(map_hbm, map_v)

    bufs = (buf0, buf1)
    sems = (sem0, sem1)

    cp0 = pltpu.async_copy(x_hbm.at[pl.ds(row0 * D, G * D)], bufs[0],
                           sems[0])

    lane = lax.iota(jnp.int32, 16)
    zeros16 = jnp.zeros((16,), jnp.int32)
    ones16 = jnp.full((16,), 1, jnp.int32)
    neg_inf = jnp.full((16,), -jnp.inf, jnp.float32)

    copies = [cp0, None]
    for g in range(NGROUPS):
        nxt = (g + 1) % 2
        if g + 1 < NGROUPS:
            copies[nxt] = pltpu.async_copy(
                x_hbm.at[pl.ds((row0 + (g + 1) * G) * D, G * D)],
                bufs[nxt], sems[nxt])
        copies[g % 2].wait()
        buf = bufs[g % 2]
        for s in range(SUB):
            base = jnp.full((16,), s * 16, jnp.int32) + lane
            row_off = base * D

            def step(j, carry):
                maxv, maxi = carry
                v = plsc.load_gather(
                    buf, [row_off + jnp.full((16,), j, jnp.int32)])
                gt = v > maxv
                maxv = jnp.where(gt, v, maxv)
                maxi = jnp.where(gt, jnp.full((16,), j, jnp.int32), maxi)
                return maxv, maxi

            _, maxi = lax.fori_loop(0, D, step, (neg_inf, zeros16))

            out_base = (jnp.full((16,), g * G + s * 16, jnp.int32)
                        + lane) * 2
            map_i = plsc.load_gather(map_v, [maxi * 2])
            map_q = plsc.load_gather(map_v, [maxi * 2 + ones16])
            plsc.store_scatter(out_v, [out_base], map_i)
            plsc.store_scatter(out_v, [out_base + ones16], map_q)

    pltpu.async_copy(out_v, out_hbm.at[pl.ds(row0 * 2, ROWS_PER_W * 2)],
                     osem).wait()


@jax.jit
def kernel(x, mapping):
    mesh = plsc.VectorSubcoreMesh(core_axis_name="c", subcore_axis_name="s")
    run = pl.kernel(
        _worker_body,
        mesh=mesh,
        compiler_params=pltpu.CompilerParams(
            use_tc_tiling_on_sc=False, needs_layout_passes=False),
        out_type=jax.ShapeDtypeStruct((B * 2,), jnp.float32),
        scratch_types=[
            pltpu.VMEM((G * D,), jnp.float32),
            pltpu.VMEM((G * D,), jnp.float32),
            pltpu.VMEM((512,), jnp.float32),
            pltpu.VMEM((ROWS_PER_W * 2,), jnp.float32),
            pltpu.SemaphoreType.DMA,
            pltpu.SemaphoreType.DMA,
            pltpu.SemaphoreType.DMA,
        ],
    )
    out = run(x.reshape(-1), mapping.reshape(-1))
    return out.reshape(B, 2)


# speedup vs baseline: 2.1291x; 2.1291x over previous
import jax
import jax.numpy as jnp
from jax import lax
from jax.experimental import pallas as pl
from jax.experimental.pallas import tpu as pltpu
from jax.experimental.pallas import tpu_sc as plsc

B = 16384
D = 256
DP = 257
NW = 32
ROWS_PER_W = B // NW
G = 64
NGROUPS = ROWS_PER_W // G
SUB = G // 16


def _worker_body(x_hbm, map_hbm, out_hbm, buf0, buf1, map_v, out_v,
                 sem0, sem1, osem):
    wid = lax.axis_index("s") * 2 + lax.axis_index("c")
    row0 = wid * ROWS_PER_W

    bufs = (buf0, buf1)
    sems = (sem0, sem1)

    cp0 = pltpu.async_copy(x_hbm.at[pl.ds(row0, G), :],
                           bufs[0].at[:, pl.ds(0, D)], sems[0])

    pltpu.sync_copy(map_hbm, map_v)

    lane = lax.iota(jnp.int32, 16)
    zeros16 = jnp.zeros((16,), jnp.int32)
    ones16 = jnp.full((16,), 1, jnp.int32)
    neg_inf = jnp.full((16,), -jnp.inf, jnp.float32)

    copies = [cp0, None]
    for g in range(NGROUPS):
        nxt = (g + 1) % 2
        if g + 1 < NGROUPS:
            copies[nxt] = pltpu.async_copy(
                x_hbm.at[pl.ds(row0 + (g + 1) * G, G), :],
                bufs[nxt].at[:, pl.ds(0, D)], sems[nxt])
        copies[g % 2].wait()
        buf = bufs[g % 2]
        for s in range(SUB):
            rows = jnp.full((16,), s * 16, jnp.int32) + lane

            @plsc.parallel_loop(0, D, unroll=8,
                                carry=(neg_inf, zeros16))
            def _cols(j, acc):
                maxv, maxi = acc
                jv = jnp.full((16,), j, jnp.int32)
                v = plsc.load_gather(buf, [rows, jv])
                gt = v > maxv
                maxv = jnp.where(gt, v, maxv)
                maxi = jnp.where(gt, jv, maxi)
                return maxv, maxi

            _, maxi = _cols

            out_rows = jnp.full((16,), g * G + s * 16, jnp.int32) + lane
            map_i = plsc.load_gather(map_v, [maxi, zeros16])
            map_q = plsc.load_gather(map_v, [maxi, ones16])
            plsc.store_scatter(out_v, [out_rows, zeros16], map_i)
            plsc.store_scatter(out_v, [out_rows, ones16], map_q)

    pltpu.async_copy(out_v, out_hbm.at[pl.ds(row0, ROWS_PER_W), :],
                     osem).wait()


@jax.jit
def kernel(x, mapping):
    mesh = plsc.VectorSubcoreMesh(core_axis_name="c", subcore_axis_name="s")
    run = pl.kernel(
        _worker_body,
        mesh=mesh,
        compiler_params=pltpu.CompilerParams(
            use_tc_tiling_on_sc=False, needs_layout_passes=False),
        out_type=jax.ShapeDtypeStruct((B, 2), jnp.float32),
        scratch_types=[
            pltpu.VMEM((G, DP), jnp.float32),
            pltpu.VMEM((G, DP), jnp.float32),
            pltpu.VMEM((256, 2), jnp.float32),
            pltpu.VMEM((ROWS_PER_W, 2), jnp.float32),
            pltpu.SemaphoreType.DMA,
            pltpu.SemaphoreType.DMA,
            pltpu.SemaphoreType.DMA,
        ],
    )
    return run(x, mapping)


# speedup vs baseline: 2.7637x; 1.2981x over previous
import jax
import jax.numpy as jnp
from jax import lax
from jax.experimental import pallas as pl
from jax.experimental.pallas import tpu as pltpu
from jax.experimental.pallas import tpu_sc as plsc

B = 16384
D = 256
TCB = 1024
NW = 32
RW = B // NW


def _argmax_block(x_ref, idx_ref):
    xb = x_ref[...]
    m = jnp.max(xb, axis=1, keepdims=True)
    io = lax.broadcasted_iota(jnp.int32, xb.shape, 1)
    masked = jnp.where(xb == m, io, jnp.int32(D - 1))
    idx_ref[...] = jnp.min(masked, axis=1)


def _lookup_body(idx_hbm, map_hbm, out_hbm, idx_v, map_v, out_v,
                 isem, osem):
    wid = lax.axis_index("s") * 2 + lax.axis_index("c")
    base = wid * RW

    cp = pltpu.async_copy(idx_hbm.at[pl.ds(base, RW)], idx_v, isem)
    pltpu.sync_copy(map_hbm, map_v)
    cp.wait()

    lane = lax.iota(jnp.int32, 16)
    ones16 = jnp.full((16,), 1, jnp.int32)
    for i in range(RW // 16):
        iv = idx_v[pl.ds(i * 16, 16)] * 2
        map_i = plsc.load_gather(map_v, [iv])
        map_q = plsc.load_gather(map_v, [iv + ones16])
        pos = (jnp.full((16,), i * 16, jnp.int32) + lane) * 2
        plsc.store_scatter(out_v, [pos], map_i)
        plsc.store_scatter(out_v, [pos + ones16], map_q)

    pltpu.async_copy(out_v, out_hbm.at[pl.ds(base * 2, RW * 2)],
                     osem).wait()


@jax.jit
def kernel(x, mapping):
    idx = pl.pallas_call(
        _argmax_block,
        grid=(B // TCB,),
        in_specs=[pl.BlockSpec((TCB, D), lambda b: (b, 0))],
        out_specs=pl.BlockSpec((TCB,), lambda b: (b,)),
        out_shape=jax.ShapeDtypeStruct((B,), jnp.int32),
    )(x)

    mesh = plsc.VectorSubcoreMesh(core_axis_name="c", subcore_axis_name="s")
    lookup = pl.kernel(
        _lookup_body,
        mesh=mesh,
        compiler_params=pltpu.CompilerParams(
            use_tc_tiling_on_sc=False, needs_layout_passes=False),
        out_type=jax.ShapeDtypeStruct((B * 2,), jnp.float32),
        scratch_types=[
            pltpu.VMEM((RW,), jnp.int32),
            pltpu.VMEM((512,), jnp.float32),
            pltpu.VMEM((RW * 2,), jnp.float32),
            pltpu.SemaphoreType.DMA,
            pltpu.SemaphoreType.DMA,
        ],
    )
    out = lookup(idx, mapping.reshape(-1))
    return out.reshape(B, 2)


# speedup vs baseline: 2.7927x; 1.0105x over previous
import jax
import jax.numpy as jnp
from jax import lax
from jax.experimental import pallas as pl
from jax.experimental.pallas import tpu as pltpu
from jax.experimental.pallas import tpu_sc as plsc

B = 16384
D = 256
TCB = 1024
NW = 32
RW = B // NW


def _argmax_block(x_ref, idx_ref):
    idx_ref[...] = jnp.argmax(x_ref[...], axis=1).astype(jnp.int32)


def _relayout_block(i_ref, q_ref, out_ref):
    n = out_ref.shape[0]
    iv = lax.broadcast_in_dim(i_ref[...], (n, 2), (0,))
    qv = lax.broadcast_in_dim(q_ref[...], (n, 2), (0,))
    col = lax.broadcasted_iota(jnp.int32, (n, 2), 1)
    out_ref[...] = jnp.where(col == 0, iv, qv)


def _lookup_body(idx_hbm, map_hbm, out_hbm, idx_v, map_v, out_v,
                 isem, osem):
    wid = lax.axis_index("s") * 2 + lax.axis_index("c")
    base = wid * RW

    cp = pltpu.async_copy(idx_hbm.at[pl.ds(base, RW)], idx_v, isem)
    pltpu.sync_copy(map_hbm, map_v)
    cp.wait()

    ones16 = jnp.full((16,), 1, jnp.int32)
    for i in range(RW // 16):
        iv = idx_v[pl.ds(i * 16, 16)] * 2
        map_i = plsc.load_gather(map_v, [iv])
        map_q = plsc.load_gather(map_v, [iv + ones16])
        out_v[pl.ds(i * 16, 16)] = map_i
        out_v[pl.ds(RW + i * 16, 16)] = map_q

    cpo_i = pltpu.async_copy(out_v.at[pl.ds(0, RW)],
                             out_hbm.at[pl.ds(base, RW)], osem)
    pltpu.async_copy(out_v.at[pl.ds(RW, RW)],
                     out_hbm.at[pl.ds(B + base, RW)], osem).wait()
    cpo_i.wait()


@jax.jit
def kernel(x, mapping):
    idx = pl.pallas_call(
        _argmax_block,
        grid=(B // TCB,),
        in_specs=[pl.BlockSpec((TCB, D), lambda b: (b, 0))],
        out_specs=pl.BlockSpec((TCB,), lambda b: (b,)),
        out_shape=jax.ShapeDtypeStruct((B,), jnp.int32),
    )(x)

    mesh = plsc.VectorSubcoreMesh(core_axis_name="c", subcore_axis_name="s")
    lookup = pl.kernel(
        _lookup_body,
        mesh=mesh,
        compiler_params=pltpu.CompilerParams(
            use_tc_tiling_on_sc=False, needs_layout_passes=False),
        out_type=jax.ShapeDtypeStruct((B * 2,), jnp.float32),
        scratch_types=[
            pltpu.VMEM((RW,), jnp.int32),
            pltpu.VMEM((512,), jnp.float32),
            pltpu.VMEM((RW * 2,), jnp.float32),
            pltpu.SemaphoreType.DMA,
            pltpu.SemaphoreType.DMA,
        ],
    )
    out1d = lookup(idx, mapping.reshape(-1))
    RB = 4096
    nb = B // RB
    return pl.pallas_call(
        _relayout_block,
        grid=(nb,),
        in_specs=[pl.BlockSpec((RB,), lambda b: (b,)),
                  pl.BlockSpec((RB,), lambda b: (b + nb,))],
        out_specs=pl.BlockSpec((RB, 2), lambda b: (b, 0)),
        out_shape=jax.ShapeDtypeStruct((B, 2), jnp.float32),
    )(out1d, out1d)


# speedup vs baseline: 3.1638x; 1.1329x over previous
import jax
import jax.numpy as jnp
from jax import lax
from jax.experimental import pallas as pl
from jax.experimental.pallas import tpu as pltpu
from jax.experimental.pallas import tpu_sc as plsc

B = 16384
D = 256
TCB = 2048
NW = 32
RW = B // NW


def _argmax_block(x_ref, idx_ref):
    xb = x_ref[...]
    m = jnp.max(xb, axis=1, keepdims=True)
    io = lax.broadcasted_iota(jnp.int32, xb.shape, 1)
    masked = jnp.where(xb == m, io, jnp.int32(D - 1))
    idx_ref[...] = jnp.min(masked, axis=1)


def _relayout_block(i_ref, q_ref, out_ref):
    n = out_ref.shape[0]
    iv = lax.broadcast_in_dim(i_ref[...], (n, 2), (0,))
    qv = lax.broadcast_in_dim(q_ref[...], (n, 2), (0,))
    col = lax.broadcasted_iota(jnp.int32, (n, 2), 1)
    out_ref[...] = jnp.where(col == 0, iv, qv)


def _lookup_body(idx_hbm, map_hbm, out_hbm, idx_v, map_v, out_v,
                 isem, osem):
    wid = lax.axis_index("s") * 2 + lax.axis_index("c")
    base = wid * RW

    cp = pltpu.async_copy(idx_hbm.at[pl.ds(base, RW)], idx_v, isem)
    pltpu.sync_copy(map_hbm, map_v)
    cp.wait()

    ones16 = jnp.full((16,), 1, jnp.int32)
    for i in range(RW // 16):
        iv = idx_v[pl.ds(i * 16, 16)] * 2
        map_i = plsc.load_gather(map_v, [iv])
        map_q = plsc.load_gather(map_v, [iv + ones16])
        out_v[pl.ds(i * 16, 16)] = map_i
        out_v[pl.ds(RW + i * 16, 16)] = map_q

    cpo_i = pltpu.async_copy(out_v.at[pl.ds(0, RW)],
                             out_hbm.at[pl.ds(base, RW)], osem)
    pltpu.async_copy(out_v.at[pl.ds(RW, RW)],
                     out_hbm.at[pl.ds(B + base, RW)], osem).wait()
    cpo_i.wait()


@jax.jit
def kernel(x, mapping):
    idx = pl.pallas_call(
        _argmax_block,
        grid=(B // TCB,),
        in_specs=[pl.BlockSpec((TCB, D), lambda b: (b, 0))],
        out_specs=pl.BlockSpec((TCB,), lambda b: (b,)),
        out_shape=jax.ShapeDtypeStruct((B,), jnp.int32),
    )(x)

    mesh = plsc.VectorSubcoreMesh(core_axis_name="c", subcore_axis_name="s")
    lookup = pl.kernel(
        _lookup_body,
        mesh=mesh,
        compiler_params=pltpu.CompilerParams(
            use_tc_tiling_on_sc=False, needs_layout_passes=False),
        out_type=jax.ShapeDtypeStruct((B * 2,), jnp.float32),
        scratch_types=[
            pltpu.VMEM((RW,), jnp.int32),
            pltpu.VMEM((512,), jnp.float32),
            pltpu.VMEM((RW * 2,), jnp.float32),
            pltpu.SemaphoreType.DMA,
            pltpu.SemaphoreType.DMA,
        ],
    )
    out1d = lookup(idx, mapping.reshape(-1))
    RB = 4096
    nb = B // RB
    return pl.pallas_call(
        _relayout_block,
        grid=(nb,),
        in_specs=[pl.BlockSpec((RB,), lambda b: (b,)),
                  pl.BlockSpec((RB,), lambda b: (b + nb,))],
        out_specs=pl.BlockSpec((RB, 2), lambda b: (b, 0)),
        out_shape=jax.ShapeDtypeStruct((B, 2), jnp.float32),
    )(out1d, out1d)


# speedup vs baseline: 3.2178x; 1.0171x over previous
import jax
import jax.numpy as jnp
from jax import lax
from jax.experimental import pallas as pl
from jax.experimental.pallas import tpu as pltpu
from jax.experimental.pallas import tpu_sc as plsc

B = 16384
D = 256
TCB = 4096
NW = 32
RW = B // NW


def _argmax_block(x_ref, idx_ref):
    xb = x_ref[...]
    m = jnp.max(xb, axis=1, keepdims=True)
    io = lax.broadcasted_iota(jnp.int32, xb.shape, 1)
    masked = jnp.where(xb == m, io, jnp.int32(D - 1))
    idx_ref[...] = jnp.min(masked, axis=1)


def _relayout_block(i_ref, q_ref, out_ref):
    n = out_ref.shape[0]
    iv = lax.broadcast_in_dim(i_ref[...], (n, 2), (0,))
    qv = lax.broadcast_in_dim(q_ref[...], (n, 2), (0,))
    col = lax.broadcasted_iota(jnp.int32, (n, 2), 1)
    out_ref[...] = jnp.where(col == 0, iv, qv)


def _lookup_body(idx_hbm, map_hbm, out_hbm, idx_v, map_v, out_v,
                 isem, osem):
    wid = lax.axis_index("s") * 2 + lax.axis_index("c")
    base = wid * RW

    cp = pltpu.async_copy(idx_hbm.at[pl.ds(base, RW)], idx_v, isem)
    pltpu.sync_copy(map_hbm, map_v)
    cp.wait()

    ones16 = jnp.full((16,), 1, jnp.int32)

    @plsc.parallel_loop(0, RW, step=16, unroll=4)
    def _rows(i):
        iv = idx_v[pl.ds(i, 16)] * 2
        map_i = plsc.load_gather(map_v, [iv])
        map_q = plsc.load_gather(map_v, [iv + ones16])
        out_v[pl.ds(i, 16)] = map_i
        out_v[pl.ds(RW + i, 16)] = map_q

    cpo_i = pltpu.async_copy(out_v.at[pl.ds(0, RW)],
                             out_hbm.at[pl.ds(base, RW)], osem)
    pltpu.async_copy(out_v.at[pl.ds(RW, RW)],
                     out_hbm.at[pl.ds(B + base, RW)], osem).wait()
    cpo_i.wait()


@jax.jit
def kernel(x, mapping):
    idx = pl.pallas_call(
        _argmax_block,
        grid=(B // TCB,),
        in_specs=[pl.BlockSpec((TCB, D), lambda b: (b, 0))],
        out_specs=pl.BlockSpec((TCB,), lambda b: (b,)),
        out_shape=jax.ShapeDtypeStruct((B,), jnp.int32),
    )(x)

    mesh = plsc.VectorSubcoreMesh(core_axis_name="c", subcore_axis_name="s")
    lookup = pl.kernel(
        _lookup_body,
        mesh=mesh,
        compiler_params=pltpu.CompilerParams(
            use_tc_tiling_on_sc=False, needs_layout_passes=False),
        out_type=jax.ShapeDtypeStruct((B * 2,), jnp.float32),
        scratch_types=[
            pltpu.VMEM((RW,), jnp.int32),
            pltpu.VMEM((512,), jnp.float32),
            pltpu.VMEM((RW * 2,), jnp.float32),
            pltpu.SemaphoreType.DMA,
            pltpu.SemaphoreType.DMA,
        ],
    )
    out1d = lookup(idx, mapping.reshape(-1))
    RB = 4096
    nb = B // RB
    return pl.pallas_call(
        _relayout_block,
        grid=(nb,),
        in_specs=[pl.BlockSpec((RB,), lambda b: (b,)),
                  pl.BlockSpec((RB,), lambda b: (b + nb,))],
        out_specs=pl.BlockSpec((RB, 2), lambda b: (b, 0)),
        out_shape=jax.ShapeDtypeStruct((B, 2), jnp.float32),
    )(out1d, out1d)
